# Initial kernel scaffold; baseline (speedup 1.0000x reference)
import jax
import jax.numpy as jnp
from jax.experimental import pallas as pl


def kernel(x):
    raise NotImplementedError("write your pallas kernel here")


# speedup vs baseline: 1.7136x; 1.7136x over previous
import functools

import jax
import jax.numpy as jnp
from jax import lax
from jax.experimental import pallas as pl
from jax.experimental.pallas import tpu as pltpu
from jax.experimental.pallas import tpu_sc as plsc

B = 8192
D = 1024
NC = 2
NS = 16
L = 16
NW = NC * NS
ROWS_PER_W = B // NW
CHUNK = 32
NCHUNK = ROWS_PER_W // CHUNK

MARK_AX, OP_AND, OP_OR, OP_XOR = 0, 1, 2, 3
ALU_LO, ALU_HI, AX_CARRY_LO, AX_CARRY_HI = 16, 32, 48, 64
OUTPUT_LO, OUTPUT_HI = 80, 96


def _decode_and_update(buf, g):
    rows = lax.iota(jnp.int32, L) + g * L

    def col(c):
        cols = jnp.full((L,), c, jnp.int32)
        return plsc.load_gather(buf, [rows, cols])

    mark = col(MARK_AX) > 0.5
    f_and = (col(OP_AND) > 0.5) & mark
    f_or = (col(OP_OR) > 0.5) & mark
    f_xor = (col(OP_XOR) > 0.5) & mark

    def argmax16(base):
        m = col(base)
        idx = jnp.zeros((L,), jnp.int32)
        for c in range(1, 16):
            v = col(base + c)
            gt = v > m
            idx = jnp.where(gt, c, idx)
            m = jnp.where(gt, v, m)
        return idx

    a = argmax16(ALU_LO) + 16 * argmax16(ALU_HI)
    b = argmax16(AX_CARRY_LO) + 16 * argmax16(AX_CARRY_HI)

    res = jnp.where(f_and, a & b, 0)
    res = jnp.where(f_or, a | b, res)
    res = jnp.where(f_xor, a ^ b, res)
    active = f_and | f_or | f_xor
    vals = jnp.where(active, 1.0, 0.0).astype(jnp.float32)

    col_lo = OUTPUT_LO + (res & 15)
    col_hi = OUTPUT_HI + (res >> 4)
    plsc.addupdate_scatter(buf, [rows, col_lo], vals)
    plsc.addupdate_scatter(buf, [rows, col_hi], vals)


@functools.partial(
    pl.kernel,
    out_type=jax.ShapeDtypeStruct((B, D), jnp.float32),
    mesh=plsc.VectorSubcoreMesh(
        core_axis_name="c", subcore_axis_name="s", num_cores=NC, num_subcores=NS
    ),
    scratch_types=[pltpu.VMEM((CHUNK, D), jnp.float32)],
    compiler_params=pltpu.CompilerParams(
        use_tc_tiling_on_sc=False, needs_layout_passes=False
    ),
)
def _sc_kernel(x_hbm, out_hbm, buf):
    wid = lax.axis_index("s") * NC + lax.axis_index("c")
    base = wid * ROWS_PER_W

    def chunk_body(k, carry):
        r0 = base + k * CHUNK
        pltpu.sync_copy(x_hbm.at[pl.ds(r0, CHUNK)], buf)
        for g in range(CHUNK // L):
            _decode_and_update(buf, g)
        pltpu.sync_copy(buf, out_hbm.at[pl.ds(r0, CHUNK)])
        return carry

    lax.fori_loop(0, NCHUNK, chunk_body, 0)


def kernel(x):
    return _sc_kernel(x)


# speedup vs baseline: 1.7862x; 1.0423x over previous
import functools

import jax
import jax.numpy as jnp
from jax import lax
from jax.experimental import pallas as pl
from jax.experimental.pallas import tpu as pltpu
from jax.experimental.pallas import tpu_sc as plsc

B = 8192
D = 1024
NC = 2
NS = 16
L = 16
NW = NC * NS
ROWS_PER_W = B // NW
CHUNK = 32
NCHUNK = ROWS_PER_W // CHUNK

MARK_AX, OP_AND, OP_OR, OP_XOR = 0, 1, 2, 3
ALU_LO, ALU_HI, AX_CARRY_LO, AX_CARRY_HI = 16, 32, 48, 64
OUTPUT_LO, OUTPUT_HI = 80, 96


def _decode_and_update(buf, g):
    rows = lax.iota(jnp.int32, L) + g * L

    def col(c):
        cols = jnp.full((L,), c, jnp.int32)
        return plsc.load_gather(buf, [rows, cols])

    mark = col(MARK_AX) > 0.5
    f_and = (col(OP_AND) > 0.5) & mark
    f_or = (col(OP_OR) > 0.5) & mark
    f_xor = (col(OP_XOR) > 0.5) & mark

    def argmax16(base):
        m = col(base)
        idx = jnp.zeros((L,), jnp.int32)
        for c in range(1, 16):
            v = col(base + c)
            gt = v > m
            idx = jnp.where(gt, c, idx)
            m = jnp.where(gt, v, m)
        return idx

    a = argmax16(ALU_LO) + 16 * argmax16(ALU_HI)
    b = argmax16(AX_CARRY_LO) + 16 * argmax16(AX_CARRY_HI)

    res = jnp.where(f_and, a & b, 0)
    res = jnp.where(f_or, a | b, res)
    res = jnp.where(f_xor, a ^ b, res)
    active = f_and | f_or | f_xor
    vals = jnp.where(active, 1.0, 0.0).astype(jnp.float32)

    col_lo = OUTPUT_LO + (res & 15)
    col_hi = OUTPUT_HI + (res >> 4)
    plsc.addupdate_scatter(buf, [rows, col_lo], vals)
    plsc.addupdate_scatter(buf, [rows, col_hi], vals)


NBUF = 3


@functools.partial(
    pl.kernel,
    out_type=jax.ShapeDtypeStruct((B, D), jnp.float32),
    mesh=plsc.VectorSubcoreMesh(
        core_axis_name="c", subcore_axis_name="s", num_cores=NC, num_subcores=NS
    ),
    scratch_types=(
        [pltpu.VMEM((CHUNK, D), jnp.float32)] * NBUF
        + [pltpu.SemaphoreType.DMA] * (2 * NBUF)
    ),
    compiler_params=pltpu.CompilerParams(
        use_tc_tiling_on_sc=False, needs_layout_passes=False
    ),
)
def _sc_kernel(x_hbm, out_hbm, *scratch):
    bufs = scratch[:NBUF]
    isems = scratch[NBUF:2 * NBUF]
    osems = scratch[2 * NBUF:]
    wid = lax.axis_index("s") * NC + lax.axis_index("c")
    base = wid * ROWS_PER_W

    def start_in(k):
        r0 = base + k * CHUNK
        return pltpu.async_copy(
            x_hbm.at[pl.ds(r0, CHUNK)], bufs[k % NBUF], isems[k % NBUF]
        )

    def start_out(k):
        r0 = base + k * CHUNK
        return pltpu.async_copy(
            bufs[k % NBUF], out_hbm.at[pl.ds(r0, CHUNK)], osems[k % NBUF]
        )

    ins = {0: start_in(0)}
    outs = {}
    for k in range(NCHUNK):
        if k + 1 < NCHUNK:
            if k >= 2:
                outs[k - 2].wait()
            ins[k + 1] = start_in(k + 1)
        ins[k].wait()
        for g in range(CHUNK // L):
            _decode_and_update(bufs[k % NBUF], g)
        outs[k] = start_out(k)
    for k in range(max(0, NCHUNK - NBUF), NCHUNK):
        outs[k].wait()


def kernel(x):
    return _sc_kernel(x)


# speedup vs baseline: 3.6547x; 2.0461x over previous
import functools

import jax
import jax.numpy as jnp
from jax import lax
from jax.experimental import pallas as pl
from jax.experimental.pallas import tpu as pltpu
from jax.experimental.pallas import tpu_sc as plsc

B = 8192
D = 1024
NC = 2
NS = 16
L = 16
NW = NC * NS
ROWS_PER_W = B // NW
CHUNK = 32
NCHUNK = ROWS_PER_W // CHUNK

MARK_AX, OP_AND, OP_OR, OP_XOR = 0, 1, 2, 3
ALU_LO, ALU_HI, AX_CARRY_LO, AX_CARRY_HI = 16, 32, 48, 64
OUTPUT_LO, OUTPUT_HI = 80, 96


def _decode_and_update(buf, g):
    rows = lax.iota(jnp.int32, L) + g * L

    def col(c):
        cols = jnp.full((L,), c, jnp.int32)
        return plsc.load_gather(buf, [rows, cols])

    mark = col(MARK_AX) > 0.5
    f_and = (col(OP_AND) > 0.5) & mark
    f_or = (col(OP_OR) > 0.5) & mark
    f_xor = (col(OP_XOR) > 0.5) & mark

    def argmax16(base):
        m = col(base)
        idx = jnp.zeros((L,), jnp.int32)
        for c in range(1, 16):
            v = col(base + c)
            gt = v > m
            idx = jnp.where(gt, c, idx)
            m = jnp.where(gt, v, m)
        return idx

    a = argmax16(ALU_LO) + 16 * argmax16(ALU_HI)
    b = argmax16(AX_CARRY_LO) + 16 * argmax16(AX_CARRY_HI)

    res = jnp.where(f_and, a & b, 0)
    res = jnp.where(f_or, a | b, res)
    res = jnp.where(f_xor, a ^ b, res)
    active = f_and | f_or | f_xor
    vals = jnp.where(active, 1.0, 0.0).astype(jnp.float32)

    col_lo = OUTPUT_LO + (res & 15)
    col_hi = OUTPUT_HI + (res >> 4)
    plsc.addupdate_scatter(buf, [rows, col_lo], vals)
    plsc.addupdate_scatter(buf, [rows, col_hi], vals)


NBUF = 3


@functools.partial(
    pl.kernel,
    out_type=jax.ShapeDtypeStruct((B, D), jnp.float32),
    mesh=plsc.VectorSubcoreMesh(
        core_axis_name="c", subcore_axis_name="s", num_cores=NC, num_subcores=NS
    ),
    scratch_types=(
        [pltpu.VMEM((CHUNK, D), jnp.float32)] * NBUF
        + [pltpu.SemaphoreType.DMA] * (2 * NBUF)
    ),
    compiler_params=pltpu.CompilerParams(needs_layout_passes=False),
)
def _sc_kernel(x_hbm, out_hbm, *scratch):
    bufs = scratch[:NBUF]
    isems = scratch[NBUF:2 * NBUF]
    osems = scratch[2 * NBUF:]
    wid = lax.axis_index("s") * NC + lax.axis_index("c")
    base = wid * ROWS_PER_W

    def start_in(k):
        r0 = base + k * CHUNK
        return pltpu.async_copy(
            x_hbm.at[pl.ds(r0, CHUNK)], bufs[k % NBUF], isems[k % NBUF]
        )

    def start_out(k):
        r0 = base + k * CHUNK
        return pltpu.async_copy(
            bufs[k % NBUF], out_hbm.at[pl.ds(r0, CHUNK)], osems[k % NBUF]
        )

    ins = {0: start_in(0)}
    outs = {}
    for k in range(NCHUNK):
        if k + 1 < NCHUNK:
            if k >= 2:
                outs[k - 2].wait()
            ins[k + 1] = start_in(k + 1)
        ins[k].wait()
        for g in range(CHUNK // L):
            _decode_and_update(bufs[k % NBUF], g)
        outs[k] = start_out(k)
    for k in range(max(0, NCHUNK - NBUF), NCHUNK):
        outs[k].wait()


def kernel(x):
    return _sc_kernel(x)
